# baseline (speedup 1.0000x reference)
import functools

import jax
import jax.numpy as jnp
from jax import lax
from jax.experimental import pallas as pl
from jax.experimental.pallas import tpu as pltpu
from jax.experimental.pallas import tpu_sc as plsc

B, P, V, D = 16384, 10, 14, 32
NC, NS, L = 2, 16, 16
NW = NC * NS
CG = 8
NCG = D // CG
BQ = 4096
NBQ = B // BQ
NTASK = P * NCG * NBQ
TPW = NTASK // NW


def _make_kernel():
    mesh = plsc.VectorSubcoreMesh(core_axis_name="c", subcore_axis_name="s")

    @functools.partial(
        pl.kernel,
        mesh=mesh,
        out_type=jax.ShapeDtypeStruct((P, D, B), jnp.float32),
        scratch_types=[
            pltpu.VMEM((V, D), jnp.float32),
            pltpu.VMEM((P, D), jnp.float32),
            pltpu.VMEM((V * P, D), jnp.float32),
            pltpu.VMEM((D,), jnp.float32),
            pltpu.VMEM((BQ,), jnp.int32),
            pltpu.VMEM((CG, BQ), jnp.float32),
            pltpu.VMEM((CG, BQ), jnp.float32),
            pltpu.SemaphoreType.DMA,
            pltpu.SemaphoreType.DMA,
        ],
        compiler_params=pltpu.CompilerParams(
            use_tc_tiling_on_sc=True, needs_layout_passes=False),
    )
    def sc_kernel(e_hbm, pe_hbm, xt_hbm, out_hbm, e_v, pe_v, tab_v,
                  tmp_v, x_v, s0, s1, w0, w1):
        wid = lax.axis_index("s") * NC + lax.axis_index("c")
        slab = (s0, s1)
        wsem = (w0, w1)

        pltpu.sync_copy(e_hbm, e_v)
        pltpu.sync_copy(pe_hbm, pe_v)

        def tab_body(v, _):
            def tab_inner(p, _):
                row = v * P + p
                for h in range(D // L):
                    tmp_v[pl.ds(h * L, L)] = (
                        e_v[v, pl.ds(h * L, L)] + pe_v[p, pl.ds(h * L, L)])
                for h in range(D // L):
                    src = (h * L + lax.iota(jnp.int32, L) - row) & (D - 1)
                    tab_v[row, pl.ds(h * L, L)] = plsc.load_gather(
                        tmp_v, [src])
                return 0

            lax.fori_loop(0, P, tab_inner, 0)
            return 0

        lax.fori_loop(0, V, tab_body, 0)

        writes = [None, None]
        for t in range(TPW):
            tid = t * NW + wid
            p = tid // (NCG * NBQ)
            rem = tid % (NCG * NBQ)
            cg = rem // NBQ
            bq = rem % NBQ
            buf = slab[t % 2]
            b0 = bq * BQ
            pltpu.sync_copy(xt_hbm.at[p, pl.ds(b0, BQ)], x_v)
            cols = [jnp.full((L,), cg * CG + r, jnp.int32) for r in range(CG)]

            if writes[t % 2] is not None:
                writes[t % 2].wait()

            @plsc.parallel_loop(0, BQ // L, unroll=8)
            def gat_body(j, buf=buf, p=p):
                iv = x_v[pl.ds(j * L, L)] * P + p
                for r in range(CG):
                    skewed = (cols[r] + iv) & (D - 1)
                    buf[r, pl.ds(j * L, L)] = plsc.load_gather(
                        tab_v, [iv, skewed])
            writes[t % 2] = pltpu.async_copy(
                buf, out_hbm.at[p, pl.ds(cg * CG, CG), pl.ds(b0, BQ)],
                wsem[t % 2])

        writes[(TPW - 2) % 2].wait()
        writes[(TPW - 1) % 2].wait()

    return sc_kernel


_sc_kernel = _make_kernel()


def kernel(x, embed_weight, pe):
    xt = x.astype(jnp.int32).T
    out_t = _sc_kernel(embed_weight, pe, xt)
    return jnp.transpose(out_t, (2, 0, 1))


# speedup vs baseline: 1.1512x; 1.1512x over previous
import functools

import jax
import jax.numpy as jnp
from jax import lax
from jax.experimental import pallas as pl
from jax.experimental.pallas import tpu as pltpu
from jax.experimental.pallas import tpu_sc as plsc

B, P, V, D = 16384, 10, 14, 32
NC, NS, L = 2, 16, 16
NW = NC * NS
CG = 8
NCG = D // CG
BQ = 4096
NBQ = B // BQ
NTASK = P * NCG * NBQ
TPW = NTASK // NW


def _make_kernel():
    mesh = plsc.VectorSubcoreMesh(core_axis_name="c", subcore_axis_name="s")

    @functools.partial(
        pl.kernel,
        mesh=mesh,
        out_type=jax.ShapeDtypeStruct((P, D, B), jnp.float32),
        scratch_types=[
            pltpu.VMEM((V, D), jnp.float32),
            pltpu.VMEM((P, D), jnp.float32),
            pltpu.VMEM((V * P, D), jnp.float32),
            pltpu.VMEM((D,), jnp.float32),
            pltpu.VMEM((BQ,), jnp.int32),
            pltpu.VMEM((BQ,), jnp.int32),
            pltpu.VMEM((CG, BQ), jnp.float32),
            pltpu.VMEM((CG, BQ), jnp.float32),
            pltpu.SemaphoreType.DMA,
            pltpu.SemaphoreType.DMA,
            pltpu.SemaphoreType.DMA,
            pltpu.SemaphoreType.DMA,
        ],
        compiler_params=pltpu.CompilerParams(
            use_tc_tiling_on_sc=True, needs_layout_passes=False),
    )
    def sc_kernel(e_hbm, pe_hbm, xt_hbm, out_hbm, e_v, pe_v, tab_v,
                  tmp_v, xv0, xv1, s0, s1, xs0, xs1, w0, w1):
        wid = lax.axis_index("s") * NC + lax.axis_index("c")
        slab = (s0, s1)
        wsem = (w0, w1)
        xv = (xv0, xv1)
        xsem = (xs0, xs1)

        def task_coords(t):
            tid = t * NW + wid
            p = tid // (NCG * NBQ)
            rem = tid % (NCG * NBQ)
            return p, rem // NBQ, (rem % NBQ) * BQ

        p0_, _, b0_ = task_coords(0)
        xcopies = [None, None]
        xcopies[0] = pltpu.async_copy(
            xt_hbm.at[p0_, pl.ds(b0_, BQ)], xv[0], xsem[0])

        pltpu.sync_copy(e_hbm, e_v)
        pltpu.sync_copy(pe_hbm, pe_v)

        def tab_body(v, _):
            def tab_inner(p, _):
                row = v * P + p
                for h in range(D // L):
                    tmp_v[pl.ds(h * L, L)] = (
                        e_v[v, pl.ds(h * L, L)] + pe_v[p, pl.ds(h * L, L)])
                for h in range(D // L):
                    src = (h * L + lax.iota(jnp.int32, L) - row) & (D - 1)
                    tab_v[row, pl.ds(h * L, L)] = plsc.load_gather(
                        tmp_v, [src])
                return 0

            lax.fori_loop(0, P, tab_inner, 0)
            return 0

        lax.fori_loop(0, V, tab_body, 0)

        writes = [None, None]
        for t in range(TPW):
            p, cg, b0 = task_coords(t)
            buf = slab[t % 2]
            x_v = xv[t % 2]
            cols = [jnp.full((L,), cg * CG + r, jnp.int32) for r in range(CG)]

            xcopies[t % 2].wait()
            if t + 1 < TPW:
                pn, _, bn = task_coords(t + 1)
                xcopies[(t + 1) % 2] = pltpu.async_copy(
                    xt_hbm.at[pn, pl.ds(bn, BQ)], xv[(t + 1) % 2],
                    xsem[(t + 1) % 2])

            if writes[t % 2] is not None:
                writes[t % 2].wait()

            @plsc.parallel_loop(0, BQ // L, unroll=4)
            def gat_body(j, buf=buf, p=p, x_v=x_v):
                iv = x_v[pl.ds(j * L, L)] * P + p
                for r in range(CG):
                    skewed = (cols[r] + iv) & (D - 1)
                    buf[r, pl.ds(j * L, L)] = plsc.load_gather(
                        tab_v, [iv, skewed])
            writes[t % 2] = pltpu.async_copy(
                buf, out_hbm.at[p, pl.ds(cg * CG, CG), pl.ds(b0, BQ)],
                wsem[t % 2])

        writes[(TPW - 2) % 2].wait()
        writes[(TPW - 1) % 2].wait()

    return sc_kernel


_sc_kernel = _make_kernel()


def kernel(x, embed_weight, pe):
    xt = x.astype(jnp.int32).T
    out_t = _sc_kernel(embed_weight, pe, xt)
    return jnp.transpose(out_t, (2, 0, 1))


# speedup vs baseline: 1.1707x; 1.0169x over previous
import functools

import jax
import jax.numpy as jnp
from jax import lax
from jax.experimental import pallas as pl
from jax.experimental.pallas import tpu as pltpu
from jax.experimental.pallas import tpu_sc as plsc

B, P, V, D = 16384, 10, 14, 32
NC, NS, L = 2, 16, 16
NW = NC * NS
CG = 8
NCG = D // CG
BQ = 4096
NBQ = B // BQ
NTASK = P * NCG * NBQ
TPW = NTASK // NW


def _make_kernel():
    mesh = plsc.VectorSubcoreMesh(core_axis_name="c", subcore_axis_name="s")

    @functools.partial(
        pl.kernel,
        mesh=mesh,
        out_type=jax.ShapeDtypeStruct((P, D, B), jnp.float32),
        scratch_types=[
            pltpu.VMEM((V, D), jnp.float32),
            pltpu.VMEM((P, D), jnp.float32),
            pltpu.VMEM((V * P, D), jnp.float32),
            pltpu.VMEM((D,), jnp.float32),
            pltpu.VMEM((BQ,), jnp.int32),
            pltpu.VMEM((BQ,), jnp.int32),
            pltpu.VMEM((CG, BQ), jnp.float32),
            pltpu.VMEM((CG, BQ), jnp.float32),
            pltpu.SemaphoreType.DMA,
            pltpu.SemaphoreType.DMA,
            pltpu.SemaphoreType.DMA,
            pltpu.SemaphoreType.DMA,
        ],
        compiler_params=pltpu.CompilerParams(
            use_tc_tiling_on_sc=True, needs_layout_passes=False),
    )
    def sc_kernel(e_hbm, pe_hbm, xt_hbm, out_hbm, e_v, pe_v, tab_v,
                  tmp_v, xv0, xv1, s0, s1, xs0, xs1, w0, w1):
        wid = lax.axis_index("s") * NC + lax.axis_index("c")
        slab = (s0, s1)
        wsem = (w0, w1)
        xv = (xv0, xv1)
        xsem = (xs0, xs1)

        def task_coords(t):
            tid = t * NW + wid
            p = tid // (NCG * NBQ)
            rem = tid % (NCG * NBQ)
            return p, rem // NBQ, (rem % NBQ) * BQ

        p0_, _, b0_ = task_coords(0)
        xcopies = [None, None]
        xcopies[0] = pltpu.async_copy(
            xt_hbm.at[p0_, pl.ds(b0_, BQ)], xv[0], xsem[0])

        pltpu.sync_copy(e_hbm, e_v)
        pltpu.sync_copy(pe_hbm, pe_v)

        def tab_body(v, _):
            def tab_inner(p, _):
                row = v * P + p
                for h in range(D // L):
                    tmp_v[pl.ds(h * L, L)] = (
                        e_v[v, pl.ds(h * L, L)] + pe_v[p, pl.ds(h * L, L)])
                for h in range(D // L):
                    src = (h * L + lax.iota(jnp.int32, L) - row) & (D - 1)
                    tab_v[row, pl.ds(h * L, L)] = plsc.load_gather(
                        tmp_v, [src])
                return 0

            lax.fori_loop(0, P, tab_inner, 0)
            return 0

        lax.fori_loop(0, V, tab_body, 0)

        writes = [None, None]
        for t in range(TPW):
            p, cg, b0 = task_coords(t)
            buf = slab[t % 2]
            x_v = xv[t % 2]
            cols = [jnp.full((L,), cg * CG + r, jnp.int32) for r in range(CG)]

            xcopies[t % 2].wait()
            if t + 1 < TPW:
                pn, _, bn = task_coords(t + 1)
                xcopies[(t + 1) % 2] = pltpu.async_copy(
                    xt_hbm.at[pn, pl.ds(bn, BQ)], xv[(t + 1) % 2],
                    xsem[(t + 1) % 2])

            if writes[t % 2] is not None:
                writes[t % 2].wait()

            @plsc.parallel_loop(0, BQ // L, unroll=2)
            def gat_body(j, buf=buf, p=p, x_v=x_v):
                iv = x_v[pl.ds(j * L, L)] * P + p
                for r in range(CG):
                    skewed = (cols[r] + iv) & (D - 1)
                    buf[r, pl.ds(j * L, L)] = plsc.load_gather(
                        tab_v, [iv, skewed])
            writes[t % 2] = pltpu.async_copy(
                buf, out_hbm.at[p, pl.ds(cg * CG, CG), pl.ds(b0, BQ)],
                wsem[t % 2])

        writes[(TPW - 2) % 2].wait()
        writes[(TPW - 1) % 2].wait()

    return sc_kernel


_sc_kernel = _make_kernel()


def kernel(x, embed_weight, pe):
    xt = x.astype(jnp.int32).T
    out_t = _sc_kernel(embed_weight, pe, xt)
    return jnp.transpose(out_t, (2, 0, 1))
